# Initial kernel scaffold; baseline (speedup 1.0000x reference)
import jax
import jax.numpy as jnp
from jax.experimental import pallas as pl


def kernel(program_features, voxel_features, cross_edge_index, W1, b1, W2, b2):
    raise NotImplementedError("write your pallas kernel here")


# speedup vs baseline: 2.5489x; 2.5489x over previous
import functools

import jax
import jax.numpy as jnp
from jax import lax
from jax.experimental import pallas as pl
from jax.experimental.pallas import tpu as pltpu
from jax.experimental.pallas import tpu_sc as plsc

N_NODE = 10000
E = 320000
D = 128
L = 16
NC = 2
NS = 16
NW = NC * NS
T = E // NW
CH = 80
NCHUNK = T // CH
ROWS_PER_TILE = 624
ROWS_REM = N_NODE - NS * ROWS_PER_TILE


def _tc_project(pf, vf, w1a, w1b, b1):
    blk = 1000

    def body(pf_ref, vf_ref, wa_ref, wb_ref, b1_ref, p_ref, v_ref):
        p_ref[...] = (
            jnp.dot(pf_ref[...], wa_ref[...], preferred_element_type=jnp.float32)
            + b1_ref[...]
        )
        v_ref[...] = jnp.dot(vf_ref[...], wb_ref[...], preferred_element_type=jnp.float32)

    return pl.pallas_call(
        body,
        grid=(N_NODE // blk,),
        in_specs=[
            pl.BlockSpec((blk, D), lambda i: (i, 0)),
            pl.BlockSpec((blk, D), lambda i: (i, 0)),
            pl.BlockSpec((D, D), lambda i: (0, 0)),
            pl.BlockSpec((D, D), lambda i: (0, 0)),
            pl.BlockSpec((1, D), lambda i: (0, 0)),
        ],
        out_specs=[
            pl.BlockSpec((blk, D), lambda i: (i, 0)),
            pl.BlockSpec((blk, D), lambda i: (i, 0)),
        ],
        out_shape=[
            jax.ShapeDtypeStruct((N_NODE, D), jnp.float32),
            jax.ShapeDtypeStruct((N_NODE, D), jnp.float32),
        ],
    )(pf, vf, w1a, w1b, b1)


def _sc_edge_partials(p, v, src, dst, w2):
    mesh = plsc.VectorSubcoreMesh(core_axis_name="c", subcore_axis_name="s")

    @functools.partial(
        pl.kernel,
        mesh=mesh,
        out_type=jax.ShapeDtypeStruct((E * L,), jnp.float32),
        scratch_types=[
            pltpu.VMEM((CH,), jnp.int32),
            pltpu.VMEM((CH,), jnp.int32),
            pltpu.VMEM((CH, D), jnp.float32),
            pltpu.VMEM((CH, D), jnp.float32),
            pltpu.VMEM((D,), jnp.float32),
            pltpu.VMEM((CH * L,), jnp.float32),
            pltpu.SemaphoreType.DMA,
            pltpu.SemaphoreType.DMA,
        ],
    )
    def k(p_hbm, v_hbm, src_hbm, dst_hbm, w2_hbm, out_hbm,
          sidx, didx, prow, vrow, w2v, part, sem1, sem2):
        wid = lax.axis_index("s") * NC + lax.axis_index("c")
        pltpu.sync_copy(w2_hbm, w2v)

        def chunk_body(c, carry):
            base = wid * T + c * CH
            pltpu.sync_copy(src_hbm.at[pl.ds(base, CH)], sidx)
            pltpu.sync_copy(dst_hbm.at[pl.ds(base, CH)], didx)
            cp1 = pltpu.async_copy(p_hbm.at[sidx], prow, sem1)
            cp2 = pltpu.async_copy(v_hbm.at[didx], vrow, sem2)
            cp1.wait()
            cp2.wait()

            def edge_body(j, ecarry):
                acc = jnp.zeros((L,), jnp.float32)
                for i in range(D // L):
                    sl = pl.ds(i * L, L)
                    x = prow[j, sl] + vrow[j, sl]
                    t = 1.0 - 2.0 / (jnp.exp(x + x) + 1.0)
                    acc = acc + t * w2v[sl]
                part[pl.ds(j * L, L)] = acc
                return ecarry

            lax.fori_loop(0, CH, edge_body, 0)
            pltpu.sync_copy(part, out_hbm.at[pl.ds(base * L, CH * L)])
            return carry

        lax.fori_loop(0, NCHUNK, chunk_body, 0)

    return k(p, v, src, dst, w2)


def _tc_group_sum(partials2d, kmat):
    blk = 2000

    def body(p_ref, k_ref, o_ref):
        o_ref[...] = jnp.dot(p_ref[...], k_ref[...], preferred_element_type=jnp.float32)

    return pl.pallas_call(
        body,
        grid=((E * L // D) // blk,),
        in_specs=[
            pl.BlockSpec((blk, D), lambda i: (i, 0)),
            pl.BlockSpec((D, D // L), lambda i: (0, 0)),
        ],
        out_specs=pl.BlockSpec((blk, D // L), lambda i: (i, 0)),
        out_shape=jax.ShapeDtypeStruct((E * L // D, D // L), jnp.float32),
    )(partials2d, kmat)


def _tc_softmax(logits2d):
    def body(x_ref, o_ref):
        x = x_ref[...]
        m = jnp.max(x)
        e = jnp.exp(x - m)
        o_ref[...] = e / jnp.sum(e)

    return pl.pallas_call(
        body,
        out_shape=jax.ShapeDtypeStruct(logits2d.shape, jnp.float32),
    )(logits2d)


def _sc_scatter(pf, src, dst, w, zeros_init):
    mesh = plsc.VectorSubcoreMesh(core_axis_name="c", subcore_axis_name="s")

    @functools.partial(
        pl.kernel,
        mesh=mesh,
        out_type=jax.ShapeDtypeStruct((NC * N_NODE, D), jnp.float32),
        scratch_types=[
            pltpu.VMEM((CH,), jnp.int32),
            pltpu.VMEM((CH,), jnp.int32),
            pltpu.VMEM((CH,), jnp.float32),
            pltpu.VMEM((CH, D), jnp.float32),
            pltpu.VMEM_SHARED((N_NODE, D), jnp.float32),
            pltpu.SemaphoreType.DMA,
        ],
    )
    def k(pf_hbm, src_hbm, dst_hbm, w_hbm, zero_hbm, out_hbm,
          sidx, didx, wbuf, rows, acc, sem):
        cid = lax.axis_index("c")
        sid = lax.axis_index("s")
        wid = sid * NC + cid
        pltpu.sync_copy(
            zero_hbm.at[pl.ds(0, ROWS_PER_TILE)],
            acc.at[pl.ds(sid * ROWS_PER_TILE, ROWS_PER_TILE)],
        )

        @pl.when(sid == NS - 1)
        def _():
            pltpu.sync_copy(
                zero_hbm.at[pl.ds(0, ROWS_REM)],
                acc.at[pl.ds(NS * ROWS_PER_TILE, ROWS_REM)],
            )

        plsc.subcore_barrier()

        def chunk_body(c, carry):
            base = wid * T + c * CH
            pltpu.sync_copy(src_hbm.at[pl.ds(base, CH)], sidx)
            pltpu.sync_copy(dst_hbm.at[pl.ds(base, CH)], didx)
            pltpu.sync_copy(w_hbm.at[pl.ds(base, CH)], wbuf)
            pltpu.async_copy(pf_hbm.at[sidx], rows, sem).wait()

            def edge_body(j, ecarry):
                grp = (j // L) * L
                wv = wbuf[pl.ds(grp, L)]
                lane = jnp.full((L,), j - grp, jnp.int32)
                wsp = wv.at[lane].get(mode="promise_in_bounds")
                for i in range(D // L):
                    sl = pl.ds(i * L, L)
                    rows[j, sl] = rows[j, sl] * wsp
                return ecarry

            lax.fori_loop(0, CH, edge_body, 0)
            pltpu.sync_copy(rows, acc.at[didx], add=True)
            return carry

        lax.fori_loop(0, NCHUNK, chunk_body, 0)
        plsc.subcore_barrier()
        pltpu.sync_copy(
            acc.at[pl.ds(sid * ROWS_PER_TILE, ROWS_PER_TILE)],
            out_hbm.at[pl.ds(cid * N_NODE + sid * ROWS_PER_TILE, ROWS_PER_TILE)],
        )

        @pl.when(sid == NS - 1)
        def _():
            pltpu.sync_copy(
                acc.at[pl.ds(NS * ROWS_PER_TILE, ROWS_REM)],
                out_hbm.at[pl.ds(cid * N_NODE + NS * ROWS_PER_TILE, ROWS_REM)],
            )

    return k(pf, src, dst, w, zeros_init)


def _tc_add(a, b):
    blk = 2000

    def body(a_ref, b_ref, o_ref):
        o_ref[...] = a_ref[...] + b_ref[...]

    return pl.pallas_call(
        body,
        grid=(N_NODE // blk,),
        in_specs=[
            pl.BlockSpec((blk, D), lambda i: (i, 0)),
            pl.BlockSpec((blk, D), lambda i: (i, 0)),
        ],
        out_specs=pl.BlockSpec((blk, D), lambda i: (i, 0)),
        out_shape=jax.ShapeDtypeStruct((N_NODE, D), jnp.float32),
    )(a, b)


def kernel(program_features, voxel_features, cross_edge_index, W1, b1, W2, b2):
    src = cross_edge_index[0].astype(jnp.int32)
    dst = cross_edge_index[1].astype(jnp.int32)
    w1a = W1[:D]
    w1b = W1[D:]
    p, v = _tc_project(
        program_features, voxel_features, w1a, w1b, b1.reshape(1, D)
    )
    w2v = W2[:, 0]
    partials = _sc_edge_partials(p, v, src, dst, w2v)
    partials2d = partials.reshape(E * L // D, D)
    kmat = (jnp.arange(D, dtype=jnp.int32)[:, None] // L
            == jnp.arange(D // L, dtype=jnp.int32)[None, :]).astype(jnp.float32)
    logits = _tc_group_sum(partials2d, kmat)
    weights2d = _tc_softmax(logits.reshape(E // D, D))
    w_flat = weights2d.reshape(E)
    zeros_init = jnp.zeros((ROWS_PER_TILE, D), jnp.float32)
    out_partials = _sc_scatter(program_features, src, dst, w_flat, zeros_init)
    output_features = _tc_add(out_partials[:N_NODE], out_partials[N_NODE:])
    attention_weights = w_flat.reshape(E, 1)
    return (output_features, attention_weights)


# speedup vs baseline: 4.7723x; 1.8723x over previous
import functools

import jax
import jax.numpy as jnp
from jax import lax
from jax.experimental import pallas as pl
from jax.experimental.pallas import tpu as pltpu
from jax.experimental.pallas import tpu_sc as plsc

N_NODE = 10000
E = 320000
D = 128
L = 16
NC = 2
NS = 16
NW = NC * NS
T = E // NW
CH = 80
NCHUNK = T // CH
ROWS_PER_TILE = 624
ROWS_REM = N_NODE - NS * ROWS_PER_TILE


def _tc_project(pf, vf, w1a, w1b, b1):
    blk = 1000

    def body(pf_ref, vf_ref, wa_ref, wb_ref, b1_ref, p_ref, v_ref):
        p_ref[...] = (
            jnp.dot(pf_ref[...], wa_ref[...], preferred_element_type=jnp.float32)
            + b1_ref[...]
        )
        v_ref[...] = jnp.dot(vf_ref[...], wb_ref[...], preferred_element_type=jnp.float32)

    return pl.pallas_call(
        body,
        grid=(N_NODE // blk,),
        in_specs=[
            pl.BlockSpec((blk, D), lambda i: (i, 0)),
            pl.BlockSpec((blk, D), lambda i: (i, 0)),
            pl.BlockSpec((D, D), lambda i: (0, 0)),
            pl.BlockSpec((D, D), lambda i: (0, 0)),
            pl.BlockSpec((1, D), lambda i: (0, 0)),
        ],
        out_specs=[
            pl.BlockSpec((blk, D), lambda i: (i, 0)),
            pl.BlockSpec((blk, D), lambda i: (i, 0)),
        ],
        out_shape=[
            jax.ShapeDtypeStruct((N_NODE, D), jnp.float32),
            jax.ShapeDtypeStruct((N_NODE, D), jnp.float32),
        ],
    )(pf, vf, w1a, w1b, b1)


def _sc_edge_partials(p, v, src, dst, w2):
    mesh = plsc.VectorSubcoreMesh(core_axis_name="c", subcore_axis_name="s")

    @functools.partial(
        pl.kernel,
        mesh=mesh,
        out_type=jax.ShapeDtypeStruct((E * L,), jnp.float32),
        scratch_types=[
            pltpu.VMEM((T,), jnp.int32),
            pltpu.VMEM((T,), jnp.int32),
            pltpu.VMEM((CH, D), jnp.float32),
            pltpu.VMEM((CH, D), jnp.float32),
            pltpu.VMEM((CH, D), jnp.float32),
            pltpu.VMEM((CH, D), jnp.float32),
            pltpu.VMEM((D,), jnp.float32),
            pltpu.VMEM((CH * L,), jnp.float32),
            pltpu.SemaphoreType.DMA,
            pltpu.SemaphoreType.DMA,
            pltpu.SemaphoreType.DMA,
            pltpu.SemaphoreType.DMA,
        ],
    )
    def k(p_hbm, v_hbm, src_hbm, dst_hbm, w2_hbm, out_hbm,
          sidx_all, didx_all, prow_a, prow_b, vrow_a, vrow_b, w2v, part,
          sp_a, sv_a, sp_b, sv_b):
        wid = lax.axis_index("s") * NC + lax.axis_index("c")
        tbase = wid * T
        pltpu.sync_copy(w2_hbm, w2v)
        pltpu.sync_copy(src_hbm.at[pl.ds(tbase, T)], sidx_all)
        pltpu.sync_copy(dst_hbm.at[pl.ds(tbase, T)], didx_all)

        def issue(c, prow, vrow, sp, sv):
            off = c * CH
            pltpu.async_copy(p_hbm.at[sidx_all.at[pl.ds(off, CH)]], prow, sp)
            pltpu.async_copy(v_hbm.at[didx_all.at[pl.ds(off, CH)]], vrow, sv)

        def wait(c, prow, vrow, sp, sv):
            off = c * CH
            pltpu.make_async_copy(p_hbm.at[sidx_all.at[pl.ds(off, CH)]], prow, sp).wait()
            pltpu.make_async_copy(v_hbm.at[didx_all.at[pl.ds(off, CH)]], vrow, sv).wait()

        def compute(c, prow, vrow):
            def edge_body(j, ecarry):
                acc = jnp.zeros((L,), jnp.float32)
                for i in range(D // L):
                    sl = pl.ds(i * L, L)
                    x = prow[j, sl] + vrow[j, sl]
                    t = 1.0 - 2.0 / (jnp.exp(x + x) + 1.0)
                    acc = acc + t * w2v[sl]
                part[pl.ds(j * L, L)] = acc
                return ecarry

            lax.fori_loop(0, CH, edge_body, 0)
            pltpu.sync_copy(part, out_hbm.at[pl.ds((tbase + c * CH) * L, CH * L)])

        issue(0, prow_a, vrow_a, sp_a, sv_a)

        def pair_body(i, carry):
            c0 = i * 2
            issue(c0 + 1, prow_b, vrow_b, sp_b, sv_b)
            wait(c0, prow_a, vrow_a, sp_a, sv_a)
            compute(c0, prow_a, vrow_a)
            issue(c0 + 2, prow_a, vrow_a, sp_a, sv_a)
            wait(c0 + 1, prow_b, vrow_b, sp_b, sv_b)
            compute(c0 + 1, prow_b, vrow_b)
            return carry

        lax.fori_loop(0, (NCHUNK - 1) // 2, pair_body, 0)
        wait(NCHUNK - 1, prow_a, vrow_a, sp_a, sv_a)
        compute(NCHUNK - 1, prow_a, vrow_a)

    return k(p, v, src, dst, w2)


def _tc_group_sum(partials2d, kmat):
    blk = 2000

    def body(p_ref, k_ref, o_ref):
        o_ref[...] = jnp.dot(p_ref[...], k_ref[...], preferred_element_type=jnp.float32)

    return pl.pallas_call(
        body,
        grid=((E * L // D) // blk,),
        in_specs=[
            pl.BlockSpec((blk, D), lambda i: (i, 0)),
            pl.BlockSpec((D, D // L), lambda i: (0, 0)),
        ],
        out_specs=pl.BlockSpec((blk, D // L), lambda i: (i, 0)),
        out_shape=jax.ShapeDtypeStruct((E * L // D, D // L), jnp.float32),
    )(partials2d, kmat)


def _tc_softmax(logits2d):
    def body(x_ref, o_ref):
        x = x_ref[...]
        m = jnp.max(x)
        e = jnp.exp(x - m)
        o_ref[...] = e / jnp.sum(e)

    return pl.pallas_call(
        body,
        out_shape=jax.ShapeDtypeStruct(logits2d.shape, jnp.float32),
    )(logits2d)


def _sc_scatter(pf, src, dst, w, zeros_init):
    mesh = plsc.VectorSubcoreMesh(core_axis_name="c", subcore_axis_name="s")

    @functools.partial(
        pl.kernel,
        mesh=mesh,
        out_type=jax.ShapeDtypeStruct((NC * N_NODE, D), jnp.float32),
        scratch_types=[
            pltpu.VMEM((T,), jnp.int32),
            pltpu.VMEM((CH,), jnp.int32),
            pltpu.VMEM((CH,), jnp.int32),
            pltpu.VMEM((CH,), jnp.float32),
            pltpu.VMEM((CH,), jnp.float32),
            pltpu.VMEM((CH, D), jnp.float32),
            pltpu.VMEM((CH, D), jnp.float32),
            pltpu.VMEM_SHARED((N_NODE, D), jnp.float32),
            pltpu.SemaphoreType.DMA,
            pltpu.SemaphoreType.DMA,
        ],
    )
    def k(pf_hbm, src_hbm, dst_hbm, w_hbm, zero_hbm, out_hbm,
          sidx_all, didx_a, didx_b, w_a, w_b, rows_a, rows_b, acc, sg_a, sg_b):
        cid = lax.axis_index("c")
        sid = lax.axis_index("s")
        wid = sid * NC + cid
        tbase = wid * T
        pltpu.sync_copy(
            zero_hbm.at[pl.ds(0, ROWS_PER_TILE)],
            acc.at[pl.ds(sid * ROWS_PER_TILE, ROWS_PER_TILE)],
        )

        @pl.when(sid == NS - 1)
        def _():
            pltpu.sync_copy(
                zero_hbm.at[pl.ds(0, ROWS_REM)],
                acc.at[pl.ds(NS * ROWS_PER_TILE, ROWS_REM)],
            )

        pltpu.sync_copy(src_hbm.at[pl.ds(tbase, T)], sidx_all)
        plsc.subcore_barrier()

        def issue(c, rows, didx, wb, sg):
            off = c * CH
            pltpu.async_copy(pf_hbm.at[sidx_all.at[pl.ds(off, CH)]], rows, sg)
            pltpu.async_copy(dst_hbm.at[pl.ds(tbase + off, CH)], didx, sg)
            pltpu.async_copy(w_hbm.at[pl.ds(tbase + off, CH)], wb, sg)

        def wait(c, rows, didx, wb, sg):
            off = c * CH
            pltpu.make_async_copy(
                pf_hbm.at[sidx_all.at[pl.ds(off, CH)]], rows, sg
            ).wait()
            pltpu.make_async_copy(dst_hbm.at[pl.ds(tbase + off, CH)], didx, sg).wait()
            pltpu.make_async_copy(w_hbm.at[pl.ds(tbase + off, CH)], wb, sg).wait()

        def process(c, rows, didx, wb):
            def edge_body(j, ecarry):
                grp = (j // L) * L
                wv = wb[pl.ds(grp, L)]
                lane = jnp.full((L,), j - grp, jnp.int32)
                wsp = wv.at[lane].get(mode="promise_in_bounds")
                for i in range(D // L):
                    sl = pl.ds(i * L, L)
                    rows[j, sl] = rows[j, sl] * wsp
                return ecarry

            lax.fori_loop(0, CH, edge_body, 0)
            pltpu.sync_copy(rows, acc.at[didx], add=True)

        issue(0, rows_a, didx_a, w_a, sg_a)

        def pair_body(i, carry):
            c0 = i * 2
            issue(c0 + 1, rows_b, didx_b, w_b, sg_b)
            wait(c0, rows_a, didx_a, w_a, sg_a)
            process(c0, rows_a, didx_a, w_a)
            issue(c0 + 2, rows_a, didx_a, w_a, sg_a)
            wait(c0 + 1, rows_b, didx_b, w_b, sg_b)
            process(c0 + 1, rows_b, didx_b, w_b)
            return carry

        lax.fori_loop(0, (NCHUNK - 1) // 2, pair_body, 0)
        wait(NCHUNK - 1, rows_a, didx_a, w_a, sg_a)
        process(NCHUNK - 1, rows_a, didx_a, w_a)

        plsc.subcore_barrier()
        pltpu.sync_copy(
            acc.at[pl.ds(sid * ROWS_PER_TILE, ROWS_PER_TILE)],
            out_hbm.at[pl.ds(cid * N_NODE + sid * ROWS_PER_TILE, ROWS_PER_TILE)],
        )

        @pl.when(sid == NS - 1)
        def _():
            pltpu.sync_copy(
                acc.at[pl.ds(NS * ROWS_PER_TILE, ROWS_REM)],
                out_hbm.at[pl.ds(cid * N_NODE + NS * ROWS_PER_TILE, ROWS_REM)],
            )

    return k(pf, src, dst, w, zeros_init)


def _tc_add(a, b):
    blk = 2000

    def body(a_ref, b_ref, o_ref):
        o_ref[...] = a_ref[...] + b_ref[...]

    return pl.pallas_call(
        body,
        grid=(N_NODE // blk,),
        in_specs=[
            pl.BlockSpec((blk, D), lambda i: (i, 0)),
            pl.BlockSpec((blk, D), lambda i: (i, 0)),
        ],
        out_specs=pl.BlockSpec((blk, D), lambda i: (i, 0)),
        out_shape=jax.ShapeDtypeStruct((N_NODE, D), jnp.float32),
    )(a, b)


def kernel(program_features, voxel_features, cross_edge_index, W1, b1, W2, b2):
    src = cross_edge_index[0].astype(jnp.int32)
    dst = cross_edge_index[1].astype(jnp.int32)
    w1a = W1[:D]
    w1b = W1[D:]
    p, v = _tc_project(
        program_features, voxel_features, w1a, w1b, b1.reshape(1, D)
    )
    w2v = W2[:, 0]
    partials = _sc_edge_partials(p, v, src, dst, w2v)
    partials2d = partials.reshape(E * L // D, D)
    kmat = (jnp.arange(D, dtype=jnp.int32)[:, None] // L
            == jnp.arange(D // L, dtype=jnp.int32)[None, :]).astype(jnp.float32)
    logits = _tc_group_sum(partials2d, kmat)
    weights2d = _tc_softmax(logits.reshape(E // D, D))
    w_flat = weights2d.reshape(E)
    zeros_init = jnp.zeros((ROWS_PER_TILE, D), jnp.float32)
    out_partials = _sc_scatter(program_features, src, dst, w_flat, zeros_init)
    output_features = _tc_add(out_partials[:N_NODE], out_partials[N_NODE:])
    attention_weights = w_flat.reshape(E, 1)
    return (output_features, attention_weights)


# speedup vs baseline: 7.0792x; 1.4834x over previous
import functools

import jax
import jax.numpy as jnp
from jax import lax
from jax.experimental import pallas as pl
from jax.experimental.pallas import tpu as pltpu
from jax.experimental.pallas import tpu_sc as plsc

N_NODE = 10000
E = 320000
D = 128
L = 16
NC = 2
NS = 16
NW = NC * NS
T = E // NW
CH = 80
NCHUNK = T // CH
ROWS_PER_TILE = 624
ROWS_REM = N_NODE - NS * ROWS_PER_TILE


def _tc_project(pf, vf, w1a, w1b, b1):
    blk = 1000

    def body(pf_ref, vf_ref, wa_ref, wb_ref, b1_ref, p_ref, v_ref):
        p_ref[...] = (
            jnp.dot(pf_ref[...], wa_ref[...], preferred_element_type=jnp.float32)
            + b1_ref[...]
        )
        v_ref[...] = jnp.dot(vf_ref[...], wb_ref[...], preferred_element_type=jnp.float32)

    return pl.pallas_call(
        body,
        grid=(N_NODE // blk,),
        in_specs=[
            pl.BlockSpec((blk, D), lambda i: (i, 0)),
            pl.BlockSpec((blk, D), lambda i: (i, 0)),
            pl.BlockSpec((D, D), lambda i: (0, 0)),
            pl.BlockSpec((D, D), lambda i: (0, 0)),
            pl.BlockSpec((1, D), lambda i: (0, 0)),
        ],
        out_specs=[
            pl.BlockSpec((blk, D), lambda i: (i, 0)),
            pl.BlockSpec((blk, D), lambda i: (i, 0)),
        ],
        out_shape=[
            jax.ShapeDtypeStruct((N_NODE, D), jnp.float32),
            jax.ShapeDtypeStruct((N_NODE, D), jnp.float32),
        ],
    )(pf, vf, w1a, w1b, b1)


def _sc_edge_partials(p, v, src, dst, w2):
    mesh = plsc.VectorSubcoreMesh(core_axis_name="c", subcore_axis_name="s")

    @functools.partial(
        pl.kernel,
        mesh=mesh,
        out_type=jax.ShapeDtypeStruct((E * L,), jnp.float32),
        scratch_types=[
            pltpu.VMEM((T,), jnp.int32),
            pltpu.VMEM((T,), jnp.int32),
            pltpu.VMEM((CH, D), jnp.float32),
            pltpu.VMEM((CH, D), jnp.float32),
            pltpu.VMEM((CH, D), jnp.float32),
            pltpu.VMEM((CH, D), jnp.float32),
            pltpu.VMEM((D,), jnp.float32),
            pltpu.VMEM((CH * L,), jnp.float32),
            pltpu.SemaphoreType.DMA,
            pltpu.SemaphoreType.DMA,
            pltpu.SemaphoreType.DMA,
            pltpu.SemaphoreType.DMA,
        ],
    )
    def k(p_hbm, v_hbm, src_hbm, dst_hbm, w2_hbm, out_hbm,
          sidx_all, didx_all, prow_a, prow_b, vrow_a, vrow_b, w2v, part,
          sp_a, sv_a, sp_b, sv_b):
        wid = lax.axis_index("s") * NC + lax.axis_index("c")
        tbase = wid * T
        pltpu.sync_copy(w2_hbm, w2v)
        pltpu.sync_copy(src_hbm.at[pl.ds(tbase, T)], sidx_all)
        pltpu.sync_copy(dst_hbm.at[pl.ds(tbase, T)], didx_all)

        def issue(c, prow, vrow, sp, sv):
            off = c * CH
            pltpu.async_copy(p_hbm.at[sidx_all.at[pl.ds(off, CH)]], prow, sp)
            pltpu.async_copy(v_hbm.at[didx_all.at[pl.ds(off, CH)]], vrow, sv)

        def wait(c, prow, vrow, sp, sv):
            off = c * CH
            pltpu.make_async_copy(p_hbm.at[sidx_all.at[pl.ds(off, CH)]], prow, sp).wait()
            pltpu.make_async_copy(v_hbm.at[didx_all.at[pl.ds(off, CH)]], vrow, sv).wait()

        def compute(c, prow, vrow):
            w2r = tuple(w2v[pl.ds(i * L, L)] for i in range(D // L))

            @plsc.parallel_loop(0, CH, 1, unroll=2, carry=w2r)
            def edge_body(j, wcar):
                acc = jnp.zeros((L,), jnp.float32)
                for i in range(D // L):
                    sl = pl.ds(i * L, L)
                    x = prow[j, sl] + vrow[j, sl]
                    t = 1.0 - 2.0 / (jnp.exp(x + x) + 1.0)
                    acc = acc + t * wcar[i]
                part[pl.ds(j * L, L)] = acc
                return wcar

            pltpu.sync_copy(part, out_hbm.at[pl.ds((tbase + c * CH) * L, CH * L)])

        issue(0, prow_a, vrow_a, sp_a, sv_a)

        def pair_body(i, carry):
            c0 = i * 2
            issue(c0 + 1, prow_b, vrow_b, sp_b, sv_b)
            wait(c0, prow_a, vrow_a, sp_a, sv_a)
            compute(c0, prow_a, vrow_a)
            issue(c0 + 2, prow_a, vrow_a, sp_a, sv_a)
            wait(c0 + 1, prow_b, vrow_b, sp_b, sv_b)
            compute(c0 + 1, prow_b, vrow_b)
            return carry

        lax.fori_loop(0, (NCHUNK - 1) // 2, pair_body, 0)
        wait(NCHUNK - 1, prow_a, vrow_a, sp_a, sv_a)
        compute(NCHUNK - 1, prow_a, vrow_a)

    return k(p, v, src, dst, w2)


def _tc_group_sum(partials2d, kmat):
    blk = 2000

    def body(p_ref, k_ref, o_ref):
        o_ref[...] = jnp.dot(p_ref[...], k_ref[...], preferred_element_type=jnp.float32)

    return pl.pallas_call(
        body,
        grid=((E * L // D) // blk,),
        in_specs=[
            pl.BlockSpec((blk, D), lambda i: (i, 0)),
            pl.BlockSpec((D, D // L), lambda i: (0, 0)),
        ],
        out_specs=pl.BlockSpec((blk, D // L), lambda i: (i, 0)),
        out_shape=jax.ShapeDtypeStruct((E * L // D, D // L), jnp.float32),
    )(partials2d, kmat)


def _tc_softmax(logits2d):
    def body(x_ref, o_ref):
        x = x_ref[...]
        m = jnp.max(x)
        e = jnp.exp(x - m)
        o_ref[...] = e / jnp.sum(e)

    return pl.pallas_call(
        body,
        out_shape=jax.ShapeDtypeStruct(logits2d.shape, jnp.float32),
    )(logits2d)


def _sc_scatter(pf, src, dst, w, zeros_init):
    mesh = plsc.VectorSubcoreMesh(core_axis_name="c", subcore_axis_name="s")

    @functools.partial(
        pl.kernel,
        mesh=mesh,
        out_type=jax.ShapeDtypeStruct((NC * N_NODE, D), jnp.float32),
        scratch_types=[
            pltpu.VMEM((T,), jnp.int32),
            pltpu.VMEM((CH,), jnp.int32),
            pltpu.VMEM((CH,), jnp.int32),
            pltpu.VMEM((CH,), jnp.float32),
            pltpu.VMEM((CH,), jnp.float32),
            pltpu.VMEM((CH, D), jnp.float32),
            pltpu.VMEM((CH, D), jnp.float32),
            pltpu.VMEM_SHARED((N_NODE, D), jnp.float32),
            pltpu.SemaphoreType.DMA,
            pltpu.SemaphoreType.DMA,
        ],
    )
    def k(pf_hbm, src_hbm, dst_hbm, w_hbm, zero_hbm, out_hbm,
          sidx_all, didx_a, didx_b, w_a, w_b, rows_a, rows_b, acc, sg_a, sg_b):
        cid = lax.axis_index("c")
        sid = lax.axis_index("s")
        wid = sid * NC + cid
        tbase = wid * T
        pltpu.sync_copy(
            zero_hbm.at[pl.ds(0, ROWS_PER_TILE)],
            acc.at[pl.ds(sid * ROWS_PER_TILE, ROWS_PER_TILE)],
        )

        @pl.when(sid == NS - 1)
        def _():
            pltpu.sync_copy(
                zero_hbm.at[pl.ds(0, ROWS_REM)],
                acc.at[pl.ds(NS * ROWS_PER_TILE, ROWS_REM)],
            )

        pltpu.sync_copy(src_hbm.at[pl.ds(tbase, T)], sidx_all)
        plsc.subcore_barrier()

        def issue(c, rows, didx, wb, sg):
            off = c * CH
            pltpu.async_copy(pf_hbm.at[sidx_all.at[pl.ds(off, CH)]], rows, sg)
            pltpu.async_copy(dst_hbm.at[pl.ds(tbase + off, CH)], didx, sg)
            pltpu.async_copy(w_hbm.at[pl.ds(tbase + off, CH)], wb, sg)

        def wait(c, rows, didx, wb, sg):
            off = c * CH
            pltpu.make_async_copy(
                pf_hbm.at[sidx_all.at[pl.ds(off, CH)]], rows, sg
            ).wait()
            pltpu.make_async_copy(dst_hbm.at[pl.ds(tbase + off, CH)], didx, sg).wait()
            pltpu.make_async_copy(w_hbm.at[pl.ds(tbase + off, CH)], wb, sg).wait()

        def process(c, rows, didx, wb):
            @plsc.parallel_loop(0, CH, 1, unroll=2)
            def edge_body(j):
                grp = (j // L) * L
                wv = wb[pl.ds(grp, L)]
                lane = jnp.full((L,), j - grp, jnp.int32)
                wsp = wv.at[lane].get(mode="promise_in_bounds")
                for i in range(D // L):
                    sl = pl.ds(i * L, L)
                    rows[j, sl] = rows[j, sl] * wsp

            pltpu.sync_copy(rows, acc.at[didx], add=True)

        issue(0, rows_a, didx_a, w_a, sg_a)

        def pair_body(i, carry):
            c0 = i * 2
            issue(c0 + 1, rows_b, didx_b, w_b, sg_b)
            wait(c0, rows_a, didx_a, w_a, sg_a)
            process(c0, rows_a, didx_a, w_a)
            issue(c0 + 2, rows_a, didx_a, w_a, sg_a)
            wait(c0 + 1, rows_b, didx_b, w_b, sg_b)
            process(c0 + 1, rows_b, didx_b, w_b)
            return carry

        lax.fori_loop(0, (NCHUNK - 1) // 2, pair_body, 0)
        wait(NCHUNK - 1, rows_a, didx_a, w_a, sg_a)
        process(NCHUNK - 1, rows_a, didx_a, w_a)

        plsc.subcore_barrier()
        pltpu.sync_copy(
            acc.at[pl.ds(sid * ROWS_PER_TILE, ROWS_PER_TILE)],
            out_hbm.at[pl.ds(cid * N_NODE + sid * ROWS_PER_TILE, ROWS_PER_TILE)],
        )

        @pl.when(sid == NS - 1)
        def _():
            pltpu.sync_copy(
                acc.at[pl.ds(NS * ROWS_PER_TILE, ROWS_REM)],
                out_hbm.at[pl.ds(cid * N_NODE + NS * ROWS_PER_TILE, ROWS_REM)],
            )

    return k(pf, src, dst, w, zeros_init)


def _tc_add(a, b):
    blk = 2000

    def body(a_ref, b_ref, o_ref):
        o_ref[...] = a_ref[...] + b_ref[...]

    return pl.pallas_call(
        body,
        grid=(N_NODE // blk,),
        in_specs=[
            pl.BlockSpec((blk, D), lambda i: (i, 0)),
            pl.BlockSpec((blk, D), lambda i: (i, 0)),
        ],
        out_specs=pl.BlockSpec((blk, D), lambda i: (i, 0)),
        out_shape=jax.ShapeDtypeStruct((N_NODE, D), jnp.float32),
    )(a, b)


def kernel(program_features, voxel_features, cross_edge_index, W1, b1, W2, b2):
    src = cross_edge_index[0].astype(jnp.int32)
    dst = cross_edge_index[1].astype(jnp.int32)
    w1a = W1[:D]
    w1b = W1[D:]
    p, v = _tc_project(
        program_features, voxel_features, w1a, w1b, b1.reshape(1, D)
    )
    w2v = W2[:, 0]
    partials = _sc_edge_partials(p, v, src, dst, w2v)
    partials2d = partials.reshape(E * L // D, D)
    kmat = (jnp.arange(D, dtype=jnp.int32)[:, None] // L
            == jnp.arange(D // L, dtype=jnp.int32)[None, :]).astype(jnp.float32)
    logits = _tc_group_sum(partials2d, kmat)
    weights2d = _tc_softmax(logits.reshape(E // D, D))
    w_flat = weights2d.reshape(E)
    zeros_init = jnp.zeros((ROWS_PER_TILE, D), jnp.float32)
    out_partials = _sc_scatter(program_features, src, dst, w_flat, zeros_init)
    output_features = _tc_add(out_partials[:N_NODE], out_partials[N_NODE:])
    attention_weights = w_flat.reshape(E, 1)
    return (output_features, attention_weights)


# speedup vs baseline: 7.4021x; 1.0456x over previous
import functools

import jax
import jax.numpy as jnp
from jax import lax
from jax.experimental import pallas as pl
from jax.experimental.pallas import tpu as pltpu
from jax.experimental.pallas import tpu_sc as plsc

N_NODE = 10000
E = 320000
D = 128
L = 16
NC = 2
NS = 16
NW = NC * NS
T = E // NW
CH = 80
NCHUNK = T // CH
ROWS_PER_TILE = 624
ROWS_REM = N_NODE - NS * ROWS_PER_TILE


def _tc_project(pf, vf, w1a, w1b, b1):
    blk = 1000

    def body(pf_ref, vf_ref, wa_ref, wb_ref, b1_ref, p_ref, v_ref):
        p_ref[...] = (
            jnp.dot(pf_ref[...], wa_ref[...], preferred_element_type=jnp.float32)
            + b1_ref[...]
        )
        v_ref[...] = jnp.dot(vf_ref[...], wb_ref[...], preferred_element_type=jnp.float32)

    return pl.pallas_call(
        body,
        grid=(N_NODE // blk,),
        in_specs=[
            pl.BlockSpec((blk, D), lambda i: (i, 0)),
            pl.BlockSpec((blk, D), lambda i: (i, 0)),
            pl.BlockSpec((D, D), lambda i: (0, 0)),
            pl.BlockSpec((D, D), lambda i: (0, 0)),
            pl.BlockSpec((1, D), lambda i: (0, 0)),
        ],
        out_specs=[
            pl.BlockSpec((blk, D), lambda i: (i, 0)),
            pl.BlockSpec((blk, D), lambda i: (i, 0)),
        ],
        out_shape=[
            jax.ShapeDtypeStruct((N_NODE, D), jnp.float32),
            jax.ShapeDtypeStruct((N_NODE, D), jnp.float32),
        ],
    )(pf, vf, w1a, w1b, b1)


def _sc_edge_partials(p, v, src, dst, w2):
    mesh = plsc.VectorSubcoreMesh(core_axis_name="c", subcore_axis_name="s")

    @functools.partial(
        pl.kernel,
        mesh=mesh,
        out_type=jax.ShapeDtypeStruct((E * L,), jnp.float32),
        scratch_types=[
            pltpu.VMEM((T,), jnp.int32),
            pltpu.VMEM((T,), jnp.int32),
            pltpu.VMEM((CH, D), jnp.float32),
            pltpu.VMEM((CH, D), jnp.float32),
            pltpu.VMEM((CH, D), jnp.float32),
            pltpu.VMEM((CH, D), jnp.float32),
            pltpu.VMEM((D,), jnp.float32),
            pltpu.VMEM((CH * L,), jnp.float32),
            pltpu.SemaphoreType.DMA,
            pltpu.SemaphoreType.DMA,
            pltpu.SemaphoreType.DMA,
            pltpu.SemaphoreType.DMA,
        ],
    )
    def k(p_hbm, v_hbm, src_hbm, dst_hbm, w2_hbm, out_hbm,
          sidx_all, didx_all, prow_a, prow_b, vrow_a, vrow_b, w2v, part,
          sp_a, sv_a, sp_b, sv_b):
        wid = lax.axis_index("s") * NC + lax.axis_index("c")
        tbase = wid * T
        pltpu.sync_copy(w2_hbm, w2v)
        pltpu.sync_copy(src_hbm.at[pl.ds(tbase, T)], sidx_all)
        pltpu.sync_copy(dst_hbm.at[pl.ds(tbase, T)], didx_all)

        def issue(c, prow, vrow, sp, sv):
            off = c * CH
            pltpu.async_copy(p_hbm.at[sidx_all.at[pl.ds(off, CH)]], prow, sp)
            pltpu.async_copy(v_hbm.at[didx_all.at[pl.ds(off, CH)]], vrow, sv)

        def wait(c, prow, vrow, sp, sv):
            off = c * CH
            pltpu.make_async_copy(p_hbm.at[sidx_all.at[pl.ds(off, CH)]], prow, sp).wait()
            pltpu.make_async_copy(v_hbm.at[didx_all.at[pl.ds(off, CH)]], vrow, sv).wait()

        def compute(c, prow, vrow):
            w2r = tuple(w2v[pl.ds(i * L, L)] for i in range(D // L))

            @plsc.parallel_loop(0, CH, 1, unroll=2, carry=w2r)
            def edge_body(j, wcar):
                acc = jnp.zeros((L,), jnp.float32)
                for i in range(D // L):
                    sl = pl.ds(i * L, L)
                    x = prow[j, sl] + vrow[j, sl]
                    t = 1.0 - 2.0 / (jnp.exp(x + x) + 1.0)
                    acc = acc + t * wcar[i]
                part[pl.ds(j * L, L)] = acc
                return wcar

            pltpu.sync_copy(part, out_hbm.at[pl.ds((tbase + c * CH) * L, CH * L)])

        issue(0, prow_a, vrow_a, sp_a, sv_a)

        def pair_body(i, carry):
            c0 = i * 2
            issue(c0 + 1, prow_b, vrow_b, sp_b, sv_b)
            wait(c0, prow_a, vrow_a, sp_a, sv_a)
            compute(c0, prow_a, vrow_a)
            issue(c0 + 2, prow_a, vrow_a, sp_a, sv_a)
            wait(c0 + 1, prow_b, vrow_b, sp_b, sv_b)
            compute(c0 + 1, prow_b, vrow_b)
            return carry

        lax.fori_loop(0, (NCHUNK - 1) // 2, pair_body, 0)
        wait(NCHUNK - 1, prow_a, vrow_a, sp_a, sv_a)
        compute(NCHUNK - 1, prow_a, vrow_a)

    return k(p, v, src, dst, w2)


def _tc_group_sum(partials2d, kmat):
    blk = 2000

    def body(p_ref, k_ref, o_ref):
        o_ref[...] = jnp.dot(p_ref[...], k_ref[...], preferred_element_type=jnp.float32)

    return pl.pallas_call(
        body,
        grid=((E * L // D) // blk,),
        in_specs=[
            pl.BlockSpec((blk, D), lambda i: (i, 0)),
            pl.BlockSpec((D, D // L), lambda i: (0, 0)),
        ],
        out_specs=pl.BlockSpec((blk, D // L), lambda i: (i, 0)),
        out_shape=jax.ShapeDtypeStruct((E * L // D, D // L), jnp.float32),
    )(partials2d, kmat)


def _tc_softmax(logits2d):
    def body(x_ref, o_ref):
        x = x_ref[...]
        m = jnp.max(x)
        e = jnp.exp(x - m)
        o_ref[...] = e / jnp.sum(e)

    return pl.pallas_call(
        body,
        out_shape=jax.ShapeDtypeStruct(logits2d.shape, jnp.float32),
    )(logits2d)


def _sc_scatter(pf, src, dst, w, zeros_init):
    mesh = plsc.VectorSubcoreMesh(core_axis_name="c", subcore_axis_name="s")

    @functools.partial(
        pl.kernel,
        mesh=mesh,
        out_type=jax.ShapeDtypeStruct((NC * N_NODE, D), jnp.float32),
        scratch_types=[
            pltpu.VMEM((T,), jnp.int32),
            pltpu.VMEM((CH,), jnp.int32),
            pltpu.VMEM((CH,), jnp.int32),
            pltpu.VMEM((CH,), jnp.int32),
            pltpu.VMEM((CH,), jnp.float32),
            pltpu.VMEM((CH,), jnp.float32),
            pltpu.VMEM((CH,), jnp.float32),
            pltpu.VMEM((CH, D), jnp.float32),
            pltpu.VMEM((CH, D), jnp.float32),
            pltpu.VMEM((CH, D), jnp.float32),
            pltpu.VMEM_SHARED((N_NODE, D), jnp.float32),
            pltpu.SemaphoreType.DMA,
            pltpu.SemaphoreType.DMA,
            pltpu.SemaphoreType.DMA,
            pltpu.SemaphoreType.DMA,
            pltpu.SemaphoreType.DMA,
            pltpu.SemaphoreType.DMA,
        ],
    )
    def k(pf_hbm, src_hbm, dst_hbm, w_hbm, zero_hbm, out_hbm,
          sidx_all, di0, di1, di2, wb0, wb1, wb2, ro0, ro1, ro2, acc,
          sg0, sg1, sg2, ss0, ss1, ss2):
        cid = lax.axis_index("c")
        sid = lax.axis_index("s")
        wid = sid * NC + cid
        tbase = wid * T
        pltpu.sync_copy(
            zero_hbm.at[pl.ds(0, ROWS_PER_TILE)],
            acc.at[pl.ds(sid * ROWS_PER_TILE, ROWS_PER_TILE)],
        )

        @pl.when(sid == NS - 1)
        def _():
            pltpu.sync_copy(
                zero_hbm.at[pl.ds(0, ROWS_REM)],
                acc.at[pl.ds(NS * ROWS_PER_TILE, ROWS_REM)],
            )

        pltpu.sync_copy(src_hbm.at[pl.ds(tbase, T)], sidx_all)
        plsc.subcore_barrier()

        slots = (
            (ro0, di0, wb0, sg0, ss0),
            (ro1, di1, wb1, sg1, ss1),
            (ro2, di2, wb2, sg2, ss2),
        )

        def issue(c, sl):
            rows, didx, wb, sg, ss = sl
            off = c * CH
            pltpu.async_copy(pf_hbm.at[sidx_all.at[pl.ds(off, CH)]], rows, sg)
            pltpu.async_copy(dst_hbm.at[pl.ds(tbase + off, CH)], didx, sg)
            pltpu.async_copy(w_hbm.at[pl.ds(tbase + off, CH)], wb, sg)

        def wait_gather(c, sl):
            rows, didx, wb, sg, ss = sl
            off = c * CH
            pltpu.make_async_copy(
                pf_hbm.at[sidx_all.at[pl.ds(off, CH)]], rows, sg
            ).wait()
            pltpu.make_async_copy(dst_hbm.at[pl.ds(tbase + off, CH)], didx, sg).wait()
            pltpu.make_async_copy(w_hbm.at[pl.ds(tbase + off, CH)], wb, sg).wait()

        def wait_scatter(sl):
            rows, didx, wb, sg, ss = sl
            pltpu.make_async_copy(rows, acc.at[didx], ss).wait()

        def process(c, sl):
            rows, didx, wb, sg, ss = sl
            wait_gather(c, sl)

            @plsc.parallel_loop(0, CH, 1, unroll=2)
            def edge_body(j):
                grp = (j // L) * L
                wv = wb[pl.ds(grp, L)]
                lane = jnp.full((L,), j - grp, jnp.int32)
                wsp = wv.at[lane].get(mode="promise_in_bounds")
                for i in range(D // L):
                    sl2 = pl.ds(i * L, L)
                    rows[j, sl2] = rows[j, sl2] * wsp

            pltpu.async_copy(rows, acc.at[didx], ss, add=True)

        issue(0, slots[0])
        issue(1, slots[1])

        def tri_body(t, carry):
            for kk in range(3):
                c = t * 3 + kk
                sl = slots[kk]
                nsl = slots[(kk + 2) % 3]

                @pl.when(c < NCHUNK)
                def _():
                    process(c, sl)

                    @pl.when(jnp.logical_and(c >= 1, c < NCHUNK - 1))
                    def _():
                        wait_scatter(nsl)

                    @pl.when(c + 2 < NCHUNK)
                    def _():
                        issue(c + 2, nsl)

            return carry

        lax.fori_loop(0, (NCHUNK + 2) // 3, tri_body, 0)
        wait_scatter(slots[(NCHUNK - 2) % 3])
        wait_scatter(slots[(NCHUNK - 1) % 3])
        plsc.subcore_barrier()
        pltpu.sync_copy(
            acc.at[pl.ds(sid * ROWS_PER_TILE, ROWS_PER_TILE)],
            out_hbm.at[pl.ds(cid * N_NODE + sid * ROWS_PER_TILE, ROWS_PER_TILE)],
        )

        @pl.when(sid == NS - 1)
        def _():
            pltpu.sync_copy(
                acc.at[pl.ds(NS * ROWS_PER_TILE, ROWS_REM)],
                out_hbm.at[pl.ds(cid * N_NODE + NS * ROWS_PER_TILE, ROWS_REM)],
            )

    return k(pf, src, dst, w, zeros_init)


def _tc_add(a, b):
    blk = 2000

    def body(a_ref, b_ref, o_ref):
        o_ref[...] = a_ref[...] + b_ref[...]

    return pl.pallas_call(
        body,
        grid=(N_NODE // blk,),
        in_specs=[
            pl.BlockSpec((blk, D), lambda i: (i, 0)),
            pl.BlockSpec((blk, D), lambda i: (i, 0)),
        ],
        out_specs=pl.BlockSpec((blk, D), lambda i: (i, 0)),
        out_shape=jax.ShapeDtypeStruct((N_NODE, D), jnp.float32),
    )(a, b)


def kernel(program_features, voxel_features, cross_edge_index, W1, b1, W2, b2):
    src = cross_edge_index[0].astype(jnp.int32)
    dst = cross_edge_index[1].astype(jnp.int32)
    w1a = W1[:D]
    w1b = W1[D:]
    p, v = _tc_project(
        program_features, voxel_features, w1a, w1b, b1.reshape(1, D)
    )
    w2v = W2[:, 0]
    partials = _sc_edge_partials(p, v, src, dst, w2v)
    partials2d = partials.reshape(E * L // D, D)
    kmat = (jnp.arange(D, dtype=jnp.int32)[:, None] // L
            == jnp.arange(D // L, dtype=jnp.int32)[None, :]).astype(jnp.float32)
    logits = _tc_group_sum(partials2d, kmat)
    weights2d = _tc_softmax(logits.reshape(E // D, D))
    w_flat = weights2d.reshape(E)
    zeros_init = jnp.zeros((ROWS_PER_TILE, D), jnp.float32)
    out_partials = _sc_scatter(program_features, src, dst, w_flat, zeros_init)
    output_features = _tc_add(out_partials[:N_NODE], out_partials[N_NODE:])
    attention_weights = w_flat.reshape(E, 1)
    return (output_features, attention_weights)


# speedup vs baseline: 7.8366x; 1.0587x over previous
import functools

import jax
import jax.numpy as jnp
from jax import lax
from jax.experimental import pallas as pl
from jax.experimental.pallas import tpu as pltpu
from jax.experimental.pallas import tpu_sc as plsc

N_NODE = 10000
E = 320000
D = 128
L = 16
NC = 2
NS = 16
NW = NC * NS
T = E // NW
CH = 80
NCHUNK = T // CH
ROWS_PER_TILE = 624
ROWS_REM = N_NODE - NS * ROWS_PER_TILE


def _tc_project(pf, vf, w1a, w1b, b1):
    blk = 1000

    def body(pf_ref, vf_ref, wa_ref, wb_ref, b1_ref, p_ref, v_ref):
        p_ref[...] = (
            jnp.dot(pf_ref[...], wa_ref[...], preferred_element_type=jnp.float32)
            + b1_ref[...]
        ) * 2.0
        v_ref[...] = (
            jnp.dot(vf_ref[...], wb_ref[...], preferred_element_type=jnp.float32) * 2.0
        )

    return pl.pallas_call(
        body,
        grid=(N_NODE // blk,),
        in_specs=[
            pl.BlockSpec((blk, D), lambda i: (i, 0)),
            pl.BlockSpec((blk, D), lambda i: (i, 0)),
            pl.BlockSpec((D, D), lambda i: (0, 0)),
            pl.BlockSpec((D, D), lambda i: (0, 0)),
            pl.BlockSpec((1, D), lambda i: (0, 0)),
        ],
        out_specs=[
            pl.BlockSpec((blk, D), lambda i: (i, 0)),
            pl.BlockSpec((blk, D), lambda i: (i, 0)),
        ],
        out_shape=[
            jax.ShapeDtypeStruct((N_NODE, D), jnp.float32),
            jax.ShapeDtypeStruct((N_NODE, D), jnp.float32),
        ],
    )(pf, vf, w1a, w1b, b1)


def _sc_edge_logits(p, v, src, dst, w2):
    mesh = plsc.VectorSubcoreMesh(core_axis_name="c", subcore_axis_name="s")

    @functools.partial(
        pl.kernel,
        mesh=mesh,
        out_type=jax.ShapeDtypeStruct((E * L,), jnp.float32),
        scratch_types=[
            pltpu.VMEM((T,), jnp.int32),
            pltpu.VMEM((T,), jnp.int32),
            pltpu.VMEM((CH, D), jnp.float32),
            pltpu.VMEM((CH, D), jnp.float32),
            pltpu.VMEM((CH, D), jnp.float32),
            pltpu.VMEM((CH, D), jnp.float32),
            pltpu.VMEM((D,), jnp.float32),
            pltpu.VMEM((CH * L,), jnp.float32),
            pltpu.SemaphoreType.DMA,
            pltpu.SemaphoreType.DMA,
            pltpu.SemaphoreType.DMA,
            pltpu.SemaphoreType.DMA,
        ],
    )
    def k(p_hbm, v_hbm, src_hbm, dst_hbm, w2_hbm, out_hbm,
          sidx_all, didx_all, prow_a, prow_b, vrow_a, vrow_b, w2v, part,
          sp_a, sv_a, sp_b, sv_b):
        wid = lax.axis_index("s") * NC + lax.axis_index("c")
        tbase = wid * T
        pltpu.sync_copy(w2_hbm, w2v)
        for i in range(D // L):
            sl = pl.ds(i * L, L)
            w2v[sl] = w2v[sl] * (-2.0)
        pltpu.sync_copy(src_hbm.at[pl.ds(tbase, T)], sidx_all)
        pltpu.sync_copy(dst_hbm.at[pl.ds(tbase, T)], didx_all)

        def issue(c, prow, vrow, sp, sv):
            off = c * CH
            pltpu.async_copy(p_hbm.at[sidx_all.at[pl.ds(off, CH)]], prow, sp)
            pltpu.async_copy(v_hbm.at[didx_all.at[pl.ds(off, CH)]], vrow, sv)

        def wait(c, prow, vrow, sp, sv):
            off = c * CH
            pltpu.make_async_copy(p_hbm.at[sidx_all.at[pl.ds(off, CH)]], prow, sp).wait()
            pltpu.make_async_copy(v_hbm.at[didx_all.at[pl.ds(off, CH)]], vrow, sv).wait()

        def compute(c, prow, vrow):
            w2r = tuple(w2v[pl.ds(i * L, L)] for i in range(D // L))

            @plsc.parallel_loop(0, CH, 1, unroll=4, carry=w2r)
            def edge_body(j, wcar):
                acc = jnp.zeros((L,), jnp.float32)
                for i in range(D // L):
                    sl = pl.ds(i * L, L)
                    x = prow[j, sl] + vrow[j, sl]
                    acc = acc + wcar[i] / (jnp.exp(x) + 1.0)
                part[pl.ds(j * L, L)] = acc
                return wcar

            pltpu.sync_copy(part, out_hbm.at[pl.ds((tbase + c * CH) * L, CH * L)])

        issue(0, prow_a, vrow_a, sp_a, sv_a)

        def pair_body(i, carry):
            c0 = i * 2
            issue(c0 + 1, prow_b, vrow_b, sp_b, sv_b)
            wait(c0, prow_a, vrow_a, sp_a, sv_a)
            compute(c0, prow_a, vrow_a)
            issue(c0 + 2, prow_a, vrow_a, sp_a, sv_a)
            wait(c0 + 1, prow_b, vrow_b, sp_b, sv_b)
            compute(c0 + 1, prow_b, vrow_b)
            return carry

        lax.fori_loop(0, (NCHUNK - 1) // 2, pair_body, 0)
        wait(NCHUNK - 1, prow_a, vrow_a, sp_a, sv_a)
        compute(NCHUNK - 1, prow_a, vrow_a)

    return k(p, v, src, dst, w2)


def _tc_group_sum(partials2d, kmat):
    blk = 2000

    def body(p_ref, k_ref, o_ref):
        o_ref[...] = jnp.dot(p_ref[...], k_ref[...], preferred_element_type=jnp.float32)

    return pl.pallas_call(
        body,
        grid=((E * L // D) // blk,),
        in_specs=[
            pl.BlockSpec((blk, D), lambda i: (i, 0)),
            pl.BlockSpec((D, D // L), lambda i: (0, 0)),
        ],
        out_specs=pl.BlockSpec((blk, D // L), lambda i: (i, 0)),
        out_shape=jax.ShapeDtypeStruct((E * L // D, D // L), jnp.float32),
    )(partials2d, kmat)


def _tc_softmax(logits2d):
    def body(x_ref, o_ref):
        x = x_ref[...]
        m = jnp.max(x)
        e = jnp.exp(x - m)
        o_ref[...] = e / jnp.sum(e)

    return pl.pallas_call(
        body,
        out_shape=jax.ShapeDtypeStruct(logits2d.shape, jnp.float32),
    )(logits2d)


def _sc_scatter(pf, src, dst, w, zeros_init):
    mesh = plsc.VectorSubcoreMesh(core_axis_name="c", subcore_axis_name="s")

    @functools.partial(
        pl.kernel,
        mesh=mesh,
        out_type=jax.ShapeDtypeStruct((NC * N_NODE, D), jnp.float32),
        scratch_types=[
            pltpu.VMEM((T,), jnp.int32),
            pltpu.VMEM((CH,), jnp.int32),
            pltpu.VMEM((CH,), jnp.int32),
            pltpu.VMEM((CH,), jnp.int32),
            pltpu.VMEM((CH,), jnp.float32),
            pltpu.VMEM((CH,), jnp.float32),
            pltpu.VMEM((CH,), jnp.float32),
            pltpu.VMEM((CH, D), jnp.float32),
            pltpu.VMEM((CH, D), jnp.float32),
            pltpu.VMEM((CH, D), jnp.float32),
            pltpu.VMEM_SHARED((N_NODE, D), jnp.float32),
            pltpu.SemaphoreType.DMA,
            pltpu.SemaphoreType.DMA,
            pltpu.SemaphoreType.DMA,
            pltpu.SemaphoreType.DMA,
            pltpu.SemaphoreType.DMA,
            pltpu.SemaphoreType.DMA,
        ],
    )
    def k(pf_hbm, src_hbm, dst_hbm, w_hbm, zero_hbm, out_hbm,
          sidx_all, di0, di1, di2, wb0, wb1, wb2, ro0, ro1, ro2, acc,
          sg0, sg1, sg2, ss0, ss1, ss2):
        cid = lax.axis_index("c")
        sid = lax.axis_index("s")
        wid = sid * NC + cid
        tbase = wid * T
        pltpu.sync_copy(
            zero_hbm.at[pl.ds(0, ROWS_PER_TILE)],
            acc.at[pl.ds(sid * ROWS_PER_TILE, ROWS_PER_TILE)],
        )

        @pl.when(sid == NS - 1)
        def _():
            pltpu.sync_copy(
                zero_hbm.at[pl.ds(0, ROWS_REM)],
                acc.at[pl.ds(NS * ROWS_PER_TILE, ROWS_REM)],
            )

        pltpu.sync_copy(src_hbm.at[pl.ds(tbase, T)], sidx_all)
        plsc.subcore_barrier()

        slots = (
            (ro0, di0, wb0, sg0, ss0),
            (ro1, di1, wb1, sg1, ss1),
            (ro2, di2, wb2, sg2, ss2),
        )

        def issue(c, sl):
            rows, didx, wb, sg, ss = sl
            off = c * CH
            pltpu.async_copy(pf_hbm.at[sidx_all.at[pl.ds(off, CH)]], rows, sg)
            pltpu.async_copy(dst_hbm.at[pl.ds(tbase + off, CH)], didx, sg)
            pltpu.async_copy(w_hbm.at[pl.ds(tbase + off, CH)], wb, sg)

        def wait_gather(c, sl):
            rows, didx, wb, sg, ss = sl
            off = c * CH
            pltpu.make_async_copy(
                pf_hbm.at[sidx_all.at[pl.ds(off, CH)]], rows, sg
            ).wait()
            pltpu.make_async_copy(dst_hbm.at[pl.ds(tbase + off, CH)], didx, sg).wait()
            pltpu.make_async_copy(w_hbm.at[pl.ds(tbase + off, CH)], wb, sg).wait()

        def wait_scatter(sl):
            rows, didx, wb, sg, ss = sl
            pltpu.make_async_copy(rows, acc.at[didx], ss).wait()

        def process(c, sl):
            rows, didx, wb, sg, ss = sl
            wait_gather(c, sl)

            @plsc.parallel_loop(0, CH, 1, unroll=2)
            def edge_body(j):
                grp = (j // L) * L
                wv = wb[pl.ds(grp, L)]
                lane = jnp.full((L,), j - grp, jnp.int32)
                wsp = wv.at[lane].get(mode="promise_in_bounds")
                for i in range(D // L):
                    sl2 = pl.ds(i * L, L)
                    rows[j, sl2] = rows[j, sl2] * wsp

            pltpu.async_copy(rows, acc.at[didx], ss, add=True)

        issue(0, slots[0])
        issue(1, slots[1])

        def tri_body(t, carry):
            for kk in range(3):
                c = t * 3 + kk
                sl = slots[kk]
                nsl = slots[(kk + 2) % 3]

                @pl.when(c < NCHUNK)
                def _():
                    process(c, sl)

                    @pl.when(jnp.logical_and(c >= 1, c < NCHUNK - 1))
                    def _():
                        wait_scatter(nsl)

                    @pl.when(c + 2 < NCHUNK)
                    def _():
                        issue(c + 2, nsl)

            return carry

        lax.fori_loop(0, (NCHUNK + 2) // 3, tri_body, 0)
        wait_scatter(slots[(NCHUNK - 2) % 3])
        wait_scatter(slots[(NCHUNK - 1) % 3])
        plsc.subcore_barrier()
        pltpu.sync_copy(
            acc.at[pl.ds(sid * ROWS_PER_TILE, ROWS_PER_TILE)],
            out_hbm.at[pl.ds(cid * N_NODE + sid * ROWS_PER_TILE, ROWS_PER_TILE)],
        )

        @pl.when(sid == NS - 1)
        def _():
            pltpu.sync_copy(
                acc.at[pl.ds(NS * ROWS_PER_TILE, ROWS_REM)],
                out_hbm.at[pl.ds(cid * N_NODE + NS * ROWS_PER_TILE, ROWS_REM)],
            )

    return k(pf, src, dst, w, zeros_init)


def _tc_add(a, b):
    blk = 2000

    def body(a_ref, b_ref, o_ref):
        o_ref[...] = a_ref[...] + b_ref[...]

    return pl.pallas_call(
        body,
        grid=(N_NODE // blk,),
        in_specs=[
            pl.BlockSpec((blk, D), lambda i: (i, 0)),
            pl.BlockSpec((blk, D), lambda i: (i, 0)),
        ],
        out_specs=pl.BlockSpec((blk, D), lambda i: (i, 0)),
        out_shape=jax.ShapeDtypeStruct((N_NODE, D), jnp.float32),
    )(a, b)


def kernel(program_features, voxel_features, cross_edge_index, W1, b1, W2, b2):
    src = cross_edge_index[0].astype(jnp.int32)
    dst = cross_edge_index[1].astype(jnp.int32)
    w1a = W1[:D]
    w1b = W1[D:]
    p, v = _tc_project(
        program_features, voxel_features, w1a, w1b, b1.reshape(1, D)
    )
    w2v = W2[:, 0]
    partials = _sc_edge_logits(p, v, src, dst, w2v)
    partials2d = partials.reshape(E * L // D, D)
    kmat = (jnp.arange(D, dtype=jnp.int32)[:, None] // L
            == jnp.arange(D // L, dtype=jnp.int32)[None, :]).astype(jnp.float32)
    logits = _tc_group_sum(partials2d, kmat)
    weights2d = _tc_softmax(logits.reshape(E // D, D))
    w_flat = weights2d.reshape(E)
    zeros_init = jnp.zeros((ROWS_PER_TILE, D), jnp.float32)
    out_partials = _sc_scatter(program_features, src, dst, w_flat, zeros_init)
    output_features = _tc_add(out_partials[:N_NODE], out_partials[N_NODE:])
    attention_weights = w_flat.reshape(E, 1)
    return (output_features, attention_weights)


# speedup vs baseline: 7.8546x; 1.0023x over previous
import functools

import jax
import jax.numpy as jnp
from jax import lax
from jax.experimental import pallas as pl
from jax.experimental.pallas import tpu as pltpu
from jax.experimental.pallas import tpu_sc as plsc

N_NODE = 10000
E = 320000
D = 128
L = 16
NC = 2
NS = 16
NW = NC * NS
T = E // NW
CH = 80
NCHUNK = T // CH
ROWS_PER_TILE = 624
ROWS_REM = N_NODE - NS * ROWS_PER_TILE


def _tc_project(pf, vf, w1a, w1b, b1):
    blk = 1000

    def body(pf_ref, vf_ref, wa_ref, wb_ref, b1_ref, p_ref, v_ref):
        p_ref[...] = (
            jnp.dot(pf_ref[...], wa_ref[...], preferred_element_type=jnp.float32)
            + b1_ref[...]
        ) * 2.0
        v_ref[...] = (
            jnp.dot(vf_ref[...], wb_ref[...], preferred_element_type=jnp.float32) * 2.0
        )

    return pl.pallas_call(
        body,
        grid=(N_NODE // blk,),
        in_specs=[
            pl.BlockSpec((blk, D), lambda i: (i, 0)),
            pl.BlockSpec((blk, D), lambda i: (i, 0)),
            pl.BlockSpec((D, D), lambda i: (0, 0)),
            pl.BlockSpec((D, D), lambda i: (0, 0)),
            pl.BlockSpec((1, D), lambda i: (0, 0)),
        ],
        out_specs=[
            pl.BlockSpec((blk, D), lambda i: (i, 0)),
            pl.BlockSpec((blk, D), lambda i: (i, 0)),
        ],
        out_shape=[
            jax.ShapeDtypeStruct((N_NODE, D), jnp.float32),
            jax.ShapeDtypeStruct((N_NODE, D), jnp.float32),
        ],
    )(pf, vf, w1a, w1b, b1)


def _sc_edge_logits(p, v, src, dst, w2):
    mesh = plsc.VectorSubcoreMesh(core_axis_name="c", subcore_axis_name="s")

    @functools.partial(
        pl.kernel,
        mesh=mesh,
        out_type=jax.ShapeDtypeStruct((E * L,), jnp.float32),
        scratch_types=[
            pltpu.VMEM((T,), jnp.int32),
            pltpu.VMEM((T,), jnp.int32),
            pltpu.VMEM((CH, D), jnp.float32),
            pltpu.VMEM((CH, D), jnp.float32),
            pltpu.VMEM((CH, D), jnp.float32),
            pltpu.VMEM((CH, D), jnp.float32),
            pltpu.VMEM((D,), jnp.float32),
            pltpu.VMEM((CH * L,), jnp.float32),
            pltpu.SemaphoreType.DMA,
            pltpu.SemaphoreType.DMA,
            pltpu.SemaphoreType.DMA,
            pltpu.SemaphoreType.DMA,
        ],
    )
    def k(p_hbm, v_hbm, src_hbm, dst_hbm, w2_hbm, out_hbm,
          sidx_all, didx_all, prow_a, prow_b, vrow_a, vrow_b, w2v, part,
          sp_a, sv_a, sp_b, sv_b):
        wid = lax.axis_index("s") * NC + lax.axis_index("c")
        tbase = wid * T
        pltpu.sync_copy(w2_hbm, w2v)
        for i in range(D // L):
            sl = pl.ds(i * L, L)
            w2v[sl] = w2v[sl] * (-2.0)
        pltpu.sync_copy(src_hbm.at[pl.ds(tbase, T)], sidx_all)
        pltpu.sync_copy(dst_hbm.at[pl.ds(tbase, T)], didx_all)

        def issue(c, prow, vrow, sp, sv):
            off = c * CH
            pltpu.async_copy(p_hbm.at[sidx_all.at[pl.ds(off, CH)]], prow, sp)
            pltpu.async_copy(v_hbm.at[didx_all.at[pl.ds(off, CH)]], vrow, sv)

        def wait(c, prow, vrow, sp, sv):
            off = c * CH
            pltpu.make_async_copy(p_hbm.at[sidx_all.at[pl.ds(off, CH)]], prow, sp).wait()
            pltpu.make_async_copy(v_hbm.at[didx_all.at[pl.ds(off, CH)]], vrow, sv).wait()

        def compute(c, prow, vrow):
            w2r = tuple(w2v[pl.ds(i * L, L)] for i in range(D // L))

            @plsc.parallel_loop(0, CH, 1, unroll=4, carry=w2r)
            def edge_body(j, wcar):
                acc = jnp.zeros((L,), jnp.float32)
                for i in range(D // L):
                    sl = pl.ds(i * L, L)
                    x = prow[j, sl] + vrow[j, sl]
                    acc = acc + wcar[i] / (jnp.exp(x) + 1.0)
                part[pl.ds(j * L, L)] = acc
                return wcar

            pltpu.sync_copy(part, out_hbm.at[pl.ds((tbase + c * CH) * L, CH * L)])

        issue(0, prow_a, vrow_a, sp_a, sv_a)

        def pair_body(i, carry):
            c0 = i * 2
            issue(c0 + 1, prow_b, vrow_b, sp_b, sv_b)
            wait(c0, prow_a, vrow_a, sp_a, sv_a)
            compute(c0, prow_a, vrow_a)
            issue(c0 + 2, prow_a, vrow_a, sp_a, sv_a)
            wait(c0 + 1, prow_b, vrow_b, sp_b, sv_b)
            compute(c0 + 1, prow_b, vrow_b)
            return carry

        lax.fori_loop(0, (NCHUNK - 1) // 2, pair_body, 0)
        wait(NCHUNK - 1, prow_a, vrow_a, sp_a, sv_a)
        compute(NCHUNK - 1, prow_a, vrow_a)

    return k(p, v, src, dst, w2)


def _tc_group_sum(partials2d, kmat):
    blk = 2000

    def body(p_ref, k_ref, o_ref):
        o_ref[...] = jnp.dot(p_ref[...], k_ref[...], preferred_element_type=jnp.float32)

    return pl.pallas_call(
        body,
        grid=((E * L // D) // blk,),
        in_specs=[
            pl.BlockSpec((blk, D), lambda i: (i, 0)),
            pl.BlockSpec((D, D // L), lambda i: (0, 0)),
        ],
        out_specs=pl.BlockSpec((blk, D // L), lambda i: (i, 0)),
        out_shape=jax.ShapeDtypeStruct((E * L // D, D // L), jnp.float32),
    )(partials2d, kmat)


def _tc_softmax(logits2d):
    def body(x_ref, o_ref):
        x = x_ref[...]
        m = jnp.max(x)
        e = jnp.exp(x - m)
        o_ref[...] = e / jnp.sum(e)

    return pl.pallas_call(
        body,
        out_shape=jax.ShapeDtypeStruct(logits2d.shape, jnp.float32),
    )(logits2d)


def _sc_scatter(pf, src, dst, w, zeros_init):
    mesh = plsc.VectorSubcoreMesh(core_axis_name="c", subcore_axis_name="s")

    @functools.partial(
        pl.kernel,
        mesh=mesh,
        out_type=jax.ShapeDtypeStruct((NC * N_NODE, D), jnp.float32),
        scratch_types=[
            pltpu.VMEM((T,), jnp.int32),
            pltpu.VMEM((CH,), jnp.int32),
            pltpu.VMEM((CH,), jnp.int32),
            pltpu.VMEM((CH,), jnp.int32),
            pltpu.VMEM((CH,), jnp.float32),
            pltpu.VMEM((CH,), jnp.float32),
            pltpu.VMEM((CH,), jnp.float32),
            pltpu.VMEM((CH, D), jnp.float32),
            pltpu.VMEM((CH, D), jnp.float32),
            pltpu.VMEM((CH, D), jnp.float32),
            pltpu.VMEM_SHARED((N_NODE, D), jnp.float32),
            pltpu.SemaphoreType.DMA,
            pltpu.SemaphoreType.DMA,
            pltpu.SemaphoreType.DMA,
            pltpu.SemaphoreType.DMA,
            pltpu.SemaphoreType.DMA,
            pltpu.SemaphoreType.DMA,
        ],
    )
    def k(pf_hbm, src_hbm, dst_hbm, w_hbm, zero_hbm, out_hbm,
          sidx_all, di0, di1, di2, wb0, wb1, wb2, ro0, ro1, ro2, acc,
          sg0, sg1, sg2, ss0, ss1, ss2):
        cid = lax.axis_index("c")
        sid = lax.axis_index("s")
        wid = sid * NC + cid
        tbase = wid * T
        pltpu.sync_copy(
            zero_hbm.at[pl.ds(0, ROWS_PER_TILE)],
            acc.at[pl.ds(sid * ROWS_PER_TILE, ROWS_PER_TILE)],
        )

        @pl.when(sid == NS - 1)
        def _():
            pltpu.sync_copy(
                zero_hbm.at[pl.ds(0, ROWS_REM)],
                acc.at[pl.ds(NS * ROWS_PER_TILE, ROWS_REM)],
            )

        pltpu.sync_copy(src_hbm.at[pl.ds(tbase, T)], sidx_all)
        plsc.subcore_barrier()

        slots = (
            (ro0, di0, wb0, sg0, ss0),
            (ro1, di1, wb1, sg1, ss1),
            (ro2, di2, wb2, sg2, ss2),
        )

        def issue(c, sl):
            rows, didx, wb, sg, ss = sl
            off = c * CH
            pltpu.async_copy(pf_hbm.at[sidx_all.at[pl.ds(off, CH)]], rows, sg)
            pltpu.async_copy(dst_hbm.at[pl.ds(tbase + off, CH)], didx, sg)
            pltpu.async_copy(w_hbm.at[pl.ds(tbase + off, CH)], wb, sg)

        def wait_gather(c, sl):
            rows, didx, wb, sg, ss = sl
            off = c * CH
            pltpu.make_async_copy(
                pf_hbm.at[sidx_all.at[pl.ds(off, CH)]], rows, sg
            ).wait()
            pltpu.make_async_copy(dst_hbm.at[pl.ds(tbase + off, CH)], didx, sg).wait()
            pltpu.make_async_copy(w_hbm.at[pl.ds(tbase + off, CH)], wb, sg).wait()

        def wait_scatter(sl):
            rows, didx, wb, sg, ss = sl
            pltpu.make_async_copy(rows, acc.at[didx], ss).wait()

        def process(c, sl):
            rows, didx, wb, sg, ss = sl
            wait_gather(c, sl)

            @plsc.parallel_loop(0, CH, 1, unroll=4)
            def edge_body(j):
                grp = (j // L) * L
                wv = wb[pl.ds(grp, L)]
                lane = jnp.full((L,), j - grp, jnp.int32)
                wsp = wv.at[lane].get(mode="promise_in_bounds")
                for i in range(D // L):
                    sl2 = pl.ds(i * L, L)
                    rows[j, sl2] = rows[j, sl2] * wsp

            pltpu.async_copy(rows, acc.at[didx], ss, add=True)

        issue(0, slots[0])
        issue(1, slots[1])

        def tri_body(t, carry):
            for kk in range(3):
                c = t * 3 + kk
                sl = slots[kk]
                nsl = slots[(kk + 2) % 3]

                @pl.when(c < NCHUNK)
                def _():
                    process(c, sl)

                    @pl.when(jnp.logical_and(c >= 1, c < NCHUNK - 1))
                    def _():
                        wait_scatter(nsl)

                    @pl.when(c + 2 < NCHUNK)
                    def _():
                        issue(c + 2, nsl)

            return carry

        lax.fori_loop(0, (NCHUNK + 2) // 3, tri_body, 0)
        wait_scatter(slots[(NCHUNK - 2) % 3])
        wait_scatter(slots[(NCHUNK - 1) % 3])
        plsc.subcore_barrier()
        pltpu.sync_copy(
            acc.at[pl.ds(sid * ROWS_PER_TILE, ROWS_PER_TILE)],
            out_hbm.at[pl.ds(cid * N_NODE + sid * ROWS_PER_TILE, ROWS_PER_TILE)],
        )

        @pl.when(sid == NS - 1)
        def _():
            pltpu.sync_copy(
                acc.at[pl.ds(NS * ROWS_PER_TILE, ROWS_REM)],
                out_hbm.at[pl.ds(cid * N_NODE + NS * ROWS_PER_TILE, ROWS_REM)],
            )

    return k(pf, src, dst, w, zeros_init)


def _tc_add(a, b):
    blk = 2000

    def body(a_ref, b_ref, o_ref):
        o_ref[...] = a_ref[...] + b_ref[...]

    return pl.pallas_call(
        body,
        grid=(N_NODE // blk,),
        in_specs=[
            pl.BlockSpec((blk, D), lambda i: (i, 0)),
            pl.BlockSpec((blk, D), lambda i: (i, 0)),
        ],
        out_specs=pl.BlockSpec((blk, D), lambda i: (i, 0)),
        out_shape=jax.ShapeDtypeStruct((N_NODE, D), jnp.float32),
    )(a, b)


def kernel(program_features, voxel_features, cross_edge_index, W1, b1, W2, b2):
    src = cross_edge_index[0].astype(jnp.int32)
    dst = cross_edge_index[1].astype(jnp.int32)
    w1a = W1[:D]
    w1b = W1[D:]
    p, v = _tc_project(
        program_features, voxel_features, w1a, w1b, b1.reshape(1, D)
    )
    w2v = W2[:, 0]
    partials = _sc_edge_logits(p, v, src, dst, w2v)
    partials2d = partials.reshape(E * L // D, D)
    kmat = (jnp.arange(D, dtype=jnp.int32)[:, None] // L
            == jnp.arange(D // L, dtype=jnp.int32)[None, :]).astype(jnp.float32)
    logits = _tc_group_sum(partials2d, kmat)
    weights2d = _tc_softmax(logits.reshape(E // D, D))
    w_flat = weights2d.reshape(E)
    zeros_init = jnp.zeros((ROWS_PER_TILE, D), jnp.float32)
    out_partials = _sc_scatter(program_features, src, dst, w_flat, zeros_init)
    output_features = _tc_add(out_partials[:N_NODE], out_partials[N_NODE:])
    attention_weights = w_flat.reshape(E, 1)
    return (output_features, attention_weights)
